# Initial kernel scaffold; baseline (speedup 1.0000x reference)
import functools

import jax
import jax.numpy as jnp
from jax import lax
from jax.experimental import pallas as pl
from jax.experimental.pallas import tpu as pltpu
from jax.experimental.pallas import tpu_sc as plsc

N = 10000
E = 640000
NTRASH = 10000
NACC = 10016
HB = 640
CH = 128
NW = 32
EPT = 20096
NCH = EPT // CH
EPAD = NW * EPT
CHA = 2512
NCHA = EPT // CHA

_mesh = plsc.VectorSubcoreMesh(core_axis_name="c", subcore_axis_name="s")


@functools.partial(
    pl.kernel,
    out_type=jax.ShapeDtypeStruct((2, 2, HB, 16), jnp.float32),
    mesh=_mesh,
    scratch_types=[
        pltpu.VMEM((CHA,), jnp.int32),
        pltpu.VMEM((HB, 16), jnp.float32),
        pltpu.VMEM((HB, 16), jnp.float32),
        pltpu.VMEM((5, CH), jnp.int32),
        pltpu.VMEM_SHARED((HB, 16), jnp.float32),
        pltpu.VMEM_SHARED((HB, 16), jnp.float32),
    ],
)
def _deg_kernel(dst1_h, dst2_h, zdeg_h, out_h,
                dstbuf, hist1, hist2, rowidx, degsh1, degsh2):
    cid = lax.axis_index("c")
    sid = lax.axis_index("s")
    pltpu.sync_copy(zdeg_h, hist1)
    pltpu.sync_copy(zdeg_h, hist2)
    stripe = pl.ds(sid * (HB // 16), HB // 16)
    pltpu.sync_copy(zdeg_h.at[pl.ds(0, HB // 16)], degsh1.at[stripe])
    pltpu.sync_copy(zdeg_h.at[pl.ds(0, HB // 16)], degsh2.at[stripe])
    iot = lax.broadcasted_iota(jnp.int32, (16,), 0)
    for cc in range(5):
        for j in range(8):
            rowidx[cc, pl.ds(j * 16, 16)] = iot + (cc * CH + j * 16)
    plsc.subcore_barrier()

    wid = sid * 2 + cid
    base = wid * EPT
    ones16 = jnp.full((16,), 1.0, jnp.float32)

    def do_set(dst_h, hist):
        def chunk(c, carry):
            off = pl.multiple_of(base + c * CHA, 8)
            pltpu.sync_copy(dst_h.at[pl.ds(off, CHA)], dstbuf)
            for j in range(CHA // 16):
                d = dstbuf[pl.ds(j * 16, 16)]
                plsc.addupdate_scatter(
                    hist,
                    [lax.shift_right_logical(d, 4), lax.bitwise_and(d, 15)],
                    ones16)
            return carry
        lax.fori_loop(0, NCHA, chunk, 0)

    do_set(dst1_h, hist1)
    do_set(dst2_h, hist2)
    for cc in range(5):
        pltpu.sync_copy(hist1.at[pl.ds(cc * CH, CH)],
                        degsh1.at[rowidx.at[cc]], add=True)
        pltpu.sync_copy(hist2.at[pl.ds(cc * CH, CH)],
                        degsh2.at[rowidx.at[cc]], add=True)
    plsc.subcore_barrier()
    pltpu.sync_copy(degsh1.at[stripe], out_h.at[0, cid, stripe])
    pltpu.sync_copy(degsh2.at[stripe], out_h.at[1, cid, stripe])


_SROWS = NACC // 16


@functools.partial(
    pl.kernel,
    out_type=jax.ShapeDtypeStruct((2, 2, NACC, 80), jnp.float32),
    mesh=_mesh,
    scratch_types=[
        pltpu.VMEM((2, CH), jnp.int32),
        pltpu.VMEM((CH, 80), jnp.float32),
        pltpu.VMEM_SHARED((NACC, 80), jnp.float32),
        pltpu.VMEM_SHARED((NACC, 80), jnp.float32),
        pltpu.SemaphoreType.DMA,
    ],
)
def _scat_kernel(hs1_h, hs2_h, e1_h, e2_h, zacc_h, out_h,
                 idxbuf, rows, acc1, acc2, gsem):
    cid = lax.axis_index("c")
    sid = lax.axis_index("s")
    r0 = sid * _SROWS
    for k in range(4):
        pltpu.sync_copy(zacc_h, acc1.at[pl.ds(r0 + k * CH, CH)])
        pltpu.sync_copy(zacc_h, acc2.at[pl.ds(r0 + k * CH, CH)])
    rem = _SROWS - 4 * CH
    pltpu.sync_copy(zacc_h.at[pl.ds(0, rem)], acc1.at[pl.ds(r0 + 4 * CH, rem)])
    pltpu.sync_copy(zacc_h.at[pl.ds(0, rem)], acc2.at[pl.ds(r0 + 4 * CH, rem)])
    plsc.subcore_barrier()

    wid = sid * 2 + cid
    base = wid * NCH

    def do_set(e_h, hs_h, acc):
        def chunk(c, carry):
            pltpu.sync_copy(e_h.at[base + c], idxbuf)
            pltpu.async_copy(hs_h.at[idxbuf.at[0]], rows, gsem).wait()
            pltpu.sync_copy(rows, acc.at[idxbuf.at[1]], add=True)
            return carry
        lax.fori_loop(0, NCH, chunk, 0)

    do_set(e1_h, hs1_h, acc1)
    do_set(e2_h, hs2_h, acc2)
    plsc.subcore_barrier()
    ostripe = pl.ds(r0, _SROWS)
    pltpu.sync_copy(acc1.at[ostripe], out_h.at[0, cid, ostripe])
    pltpu.sync_copy(acc2.at[ostripe], out_h.at[1, cid, ostripe])


def _cnn_body(x_ref, wfc_ref, bfc_ref, wl_ref, bl_ref, o_ref):
    xb = x_ref[...]
    y = xb[:, 2:3]
    xf = xb[:, 3:]
    h1 = jnp.maximum(
        jnp.dot(xf, wfc_ref[...], preferred_element_type=jnp.float32)
        + bfc_ref[...], 0.0)
    h2 = jnp.maximum(
        jnp.dot(h1, wl_ref[...], preferred_element_type=jnp.float32)
        + bl_ref[...], 0.0)
    o_ref[...] = jnp.concatenate([h2, y], axis=1)


def _gru_body(xg_ref, wih_ref, whh_ref, bih_ref, bhh_ref, o_ref):
    xg = xg_ref[...]
    bn = xg.shape[0]
    h = jnp.zeros((bn, 64), jnp.float32)
    for t in range(4):
        xt = xg[:, t * 41:(t + 1) * 41]
        gi = jnp.dot(xt, wih_ref[...],
                     preferred_element_type=jnp.float32) + bih_ref[...]
        gh = jnp.dot(h, whh_ref[...],
                     preferred_element_type=jnp.float32) + bhh_ref[...]
        r = jax.nn.sigmoid(gi[:, :64] + gh[:, :64])
        z = jax.nn.sigmoid(gi[:, 64:128] + gh[:, 64:128])
        nn_ = jnp.tanh(gi[:, 128:] + r * gh[:, 128:])
        h = (1.0 - z) * nn_ + z * h
    o_ref[...] = jnp.concatenate([xg[:, 164:204], h], axis=1)


def _h_body(x0_ref, degs_ref, w_ref, hs1_ref, hs2_ref, dinv_ref):
    dinv = lax.rsqrt(degs_ref[...] + 1.0)
    hb = jnp.dot(x0_ref[...], w_ref[...],
                 preferred_element_type=jnp.float32)
    hs1_ref[...] = hb[:, :80] * dinv[:, 0:1]
    hs2_ref[...] = hb[:, 40:] * dinv[:, 1:2]
    dinv_ref[...] = dinv


def _final_body(acc_ref, hs1_ref, hs2_ref, dinv_ref,
                bs1_ref, bc_ref, bs2_ref, aw1_ref, ab1_ref, aw2_ref,
                wm_ref, bm_ref,
                out_ref, beta_ref, e1_ref, c1_ref, c2_ref, e2_ref, emb_ref):
    av = acc_ref[...]
    dinv = dinv_ref[...]
    o1 = (av[0] + av[1] + hs1_ref[...]) * dinv[:, 0:1]
    o2 = (av[2] + av[3] + hs2_ref[...]) * dinv[:, 1:2]
    emb1 = jnp.maximum(o1[:, :40] + bs1_ref[...], 0.0)
    com1 = jnp.maximum(o1[:, 40:] + bc_ref[...], 0.0)
    com2 = jnp.maximum(o2[:, :40] + bc_ref[...], 0.0)
    emb2 = jnp.maximum(o2[:, 40:] + bs2_ref[...], 0.0)
    xcom = (com1 + com2) * 0.5
    aw2 = aw2_ref[...]
    ws = []
    for zb in (emb1, emb2, xcom):
        t1 = jnp.tanh(jnp.dot(zb, aw1_ref[...],
                              preferred_element_type=jnp.float32)
                      + ab1_ref[...])
        ws.append(jnp.sum(t1 * aw2, axis=1, keepdims=True))
    w = jnp.concatenate(ws, axis=1)
    wmax = jnp.max(w, axis=1, keepdims=True)
    ew = jnp.exp(w - wmax)
    beta = ew / jnp.sum(ew, axis=1, keepdims=True)
    emb = (beta[:, 0:1] * emb1 + beta[:, 1:2] * emb2 + beta[:, 2:3] * xcom)
    out_ref[...] = (jnp.sum(emb * wm_ref[...], axis=1, keepdims=True)
                    + bm_ref[...])
    beta_ref[...] = beta
    e1_ref[...] = emb1
    c1_ref[...] = com1
    c2_ref[...] = com2
    e2_ref[...] = emb2
    emb_ref[...] = emb


def _row_spec(bn, cols):
    return pl.BlockSpec((bn, cols), lambda i: (i, 0))


def _whole(shape):
    return pl.BlockSpec(shape, lambda i: tuple(0 for _ in shape))


@jax.jit
def kernel(x, edge_index, feat_edge_index, W_fc, b_fc, W_lin1, b_lin1,
           W_ih, W_hh, b_ih, b_hh, W_s1, b_s1, W_s2, b_s2, W_c, b_c,
           att_W1, att_b1, att_W2, W_mlp, b_mlp):
    f32 = jnp.float32

    x2d = x.reshape(N * 5, 395)
    R = 2000
    xcat = pl.pallas_call(
        _cnn_body,
        grid=(N * 5 // R,),
        in_specs=[_row_spec(R, 395), _whole((392, 80)), _whole((1, 80)),
                  _whole((80, 40)), _whole((1, 40))],
        out_specs=_row_spec(R, 41),
        out_shape=jax.ShapeDtypeStruct((N * 5, 41), f32),
    )(x2d, W_fc, b_fc.reshape(1, 80), W_lin1, b_lin1.reshape(1, 40))

    xg = xcat.reshape(N, 205)
    Bn = 2000
    x0 = pl.pallas_call(
        _gru_body,
        grid=(N // Bn,),
        in_specs=[_row_spec(Bn, 205), _whole((41, 192)), _whole((64, 192)),
                  _whole((1, 192)), _whole((1, 192))],
        out_specs=_row_spec(Bn, 104),
        out_shape=jax.ShapeDtypeStruct((N, 104), f32),
    )(xg, W_ih.T, W_hh.T, b_ih.reshape(1, 192), b_hh.reshape(1, 192))

    ei = edge_index.astype(jnp.int32)
    fei = feat_edge_index.astype(jnp.int32)
    pad_src = jnp.zeros((EPAD - E,), jnp.int32)
    pad_dst = jnp.full((EPAD - E,), NTRASH, jnp.int32)

    def prep(e):
        src = jnp.concatenate([e[0], pad_src]).reshape(EPAD // CH, 1, CH)
        dst = jnp.concatenate([e[1], pad_dst]).reshape(EPAD // CH, 1, CH)
        return jnp.concatenate([src, dst], axis=1), dst.reshape(EPAD)

    e1, dst1 = prep(ei)
    e2, dst2 = prep(fei)

    zdeg = jnp.zeros((HB, 16), f32)
    degp = _deg_kernel(dst1, dst2, zdeg)
    degs = degp.reshape(2, 2, HB * 16).sum(axis=1)[:, :N].T

    Wcat = jnp.concatenate([W_s1, W_c, W_s2], axis=1)
    hs1, hs2, dinvs = pl.pallas_call(
        _h_body,
        grid=(N // Bn,),
        in_specs=[_row_spec(Bn, 104), _row_spec(Bn, 2), _whole((104, 120))],
        out_specs=[_row_spec(Bn, 80), _row_spec(Bn, 80), _row_spec(Bn, 2)],
        out_shape=[jax.ShapeDtypeStruct((N, 80), f32),
                   jax.ShapeDtypeStruct((N, 80), f32),
                   jax.ShapeDtypeStruct((N, 2), f32)],
    )(x0, degs, Wcat)

    zacc = jnp.zeros((CH, 80), f32)
    accp = _scat_kernel(hs1, hs2, e1, e2, zacc)
    acc4 = accp.reshape(4, NACC, 80)[:, :N, :]

    accspec = pl.BlockSpec((4, Bn, 80), lambda i: (0, i, 0))
    outs = pl.pallas_call(
        _final_body,
        grid=(N // Bn,),
        in_specs=[accspec, _row_spec(Bn, 80), _row_spec(Bn, 80),
                  _row_spec(Bn, 2), _whole((1, 40)), _whole((1, 40)),
                  _whole((1, 40)), _whole((40, 16)), _whole((1, 16)),
                  _whole((1, 16)), _whole((1, 40)), _whole((1, 1))],
        out_specs=[_row_spec(Bn, 1), _row_spec(Bn, 3), _row_spec(Bn, 40),
                   _row_spec(Bn, 40), _row_spec(Bn, 40), _row_spec(Bn, 40),
                   _row_spec(Bn, 40)],
        out_shape=[jax.ShapeDtypeStruct((N, 1), f32),
                   jax.ShapeDtypeStruct((N, 3), f32),
                   jax.ShapeDtypeStruct((N, 40), f32),
                   jax.ShapeDtypeStruct((N, 40), f32),
                   jax.ShapeDtypeStruct((N, 40), f32),
                   jax.ShapeDtypeStruct((N, 40), f32),
                   jax.ShapeDtypeStruct((N, 40), f32)],
    )(acc4, hs1, hs2, dinvs,
      b_s1.reshape(1, 40), b_c.reshape(1, 40), b_s2.reshape(1, 40),
      att_W1, att_b1.reshape(1, 16), att_W2.reshape(1, 16),
      W_mlp.reshape(1, 40), b_mlp.reshape(1, 1))
    output, beta, emb1, com1, com2, emb2, emb = outs
    return (output, beta.reshape(N, 3, 1), emb1, com1, com2, emb2, emb)


# speedup vs baseline: 18.5175x; 18.5175x over previous
import functools

import jax
import jax.numpy as jnp
from jax import lax
from jax.experimental import pallas as pl
from jax.experimental.pallas import tpu as pltpu
from jax.experimental.pallas import tpu_sc as plsc

N = 10000
E = 640000
NTRASH = 10000
NACC = 10240
CH = 128
NW = 32
EPT = 20096
NCH = EPT // CH
EPAD = NW * EPT
WD = 8


@functools.lru_cache(maxsize=None)
def _sc_mesh():
    return plsc.VectorSubcoreMesh(core_axis_name="c", subcore_axis_name="s")


@functools.lru_cache(maxsize=None)
def _deg_kernel_fn():
    return functools.partial(
        pl.kernel,
        out_type=jax.ShapeDtypeStruct((2, 2, NACC, WD), jnp.float32),
        mesh=_sc_mesh(),
        scratch_types=[
            pltpu.VMEM((CH,), jnp.int32),
            pltpu.VMEM((CH, WD), jnp.float32),
            pltpu.VMEM_SHARED((NACC, WD), jnp.float32),
            pltpu.VMEM_SHARED((NACC, WD), jnp.float32),
        ],
        compiler_params=pltpu.CompilerParams(use_tc_tiling_on_sc=False),
    )(_deg_body)


def _deg_body(dst1_h, dst2_h, zdeg_h, ones_h, out_h,
              dstbuf, ones_v, dacc1, dacc2):
    cid = lax.axis_index("c")
    sid = lax.axis_index("s")
    r0 = sid * _SROWS
    for k in range(_SROWS // CH):
        pltpu.sync_copy(zdeg_h, dacc1.at[pl.ds(r0 + k * CH, CH)])
        pltpu.sync_copy(zdeg_h, dacc2.at[pl.ds(r0 + k * CH, CH)])
    pltpu.sync_copy(ones_h, ones_v)
    plsc.subcore_barrier()

    wid = sid * 2 + cid
    base = wid * EPT

    def do_set(dst_h, dacc):
        def chunk(c, carry):
            off = pl.multiple_of(base + c * CH, 8)
            pltpu.sync_copy(dst_h.at[pl.ds(off, CH)], dstbuf)
            pltpu.sync_copy(ones_v, dacc.at[dstbuf], add=True)
            return carry
        lax.fori_loop(0, NCH, chunk, 0)

    do_set(dst1_h, dacc1)
    do_set(dst2_h, dacc2)
    plsc.subcore_barrier()
    ostripe = pl.ds(r0, _SROWS)
    pltpu.sync_copy(dacc1.at[ostripe], out_h.at[0, cid, ostripe])
    pltpu.sync_copy(dacc2.at[ostripe], out_h.at[1, cid, ostripe])


_SROWS = NACC // 16


@functools.lru_cache(maxsize=None)
def _scat_kernel_fn():
    return functools.partial(
        pl.kernel,
        out_type=jax.ShapeDtypeStruct((2, 2, NACC, 80), jnp.float32),
        mesh=_sc_mesh(),
        scratch_types=[
            pltpu.VMEM((CH,), jnp.int32),
            pltpu.VMEM((CH,), jnp.int32),
            pltpu.VMEM((CH, 80), jnp.float32),
            pltpu.VMEM_SHARED((NACC, 80), jnp.float32),
            pltpu.VMEM_SHARED((NACC, 80), jnp.float32),
            pltpu.SemaphoreType.DMA,
        ],
        compiler_params=pltpu.CompilerParams(use_tc_tiling_on_sc=False),
    )(_scat_body)


def _scat_body(hs1_h, hs2_h, src1_h, dst1_h, src2_h, dst2_h, zacc_h, out_h,
               srcbuf, dstbuf, rows, acc1, acc2, gsem):
    cid = lax.axis_index("c")
    sid = lax.axis_index("s")
    r0 = sid * _SROWS
    for k in range(_SROWS // CH):
        pltpu.sync_copy(zacc_h, acc1.at[pl.ds(r0 + k * CH, CH)])
        pltpu.sync_copy(zacc_h, acc2.at[pl.ds(r0 + k * CH, CH)])
    plsc.subcore_barrier()

    wid = sid * 2 + cid
    base = wid * EPT

    def do_set(src_h, dst_h, hs_h, acc):
        def chunk(c, carry):
            off = pl.multiple_of(base + c * CH, 8)
            pltpu.sync_copy(src_h.at[pl.ds(off, CH)], srcbuf)
            pltpu.sync_copy(dst_h.at[pl.ds(off, CH)], dstbuf)
            pltpu.async_copy(hs_h.at[srcbuf], rows, gsem).wait()
            pltpu.sync_copy(rows, acc.at[dstbuf], add=True)
            return carry
        lax.fori_loop(0, NCH, chunk, 0)

    do_set(src1_h, dst1_h, hs1_h, acc1)
    do_set(src2_h, dst2_h, hs2_h, acc2)
    plsc.subcore_barrier()
    ostripe = pl.ds(r0, _SROWS)
    pltpu.sync_copy(acc1.at[ostripe], out_h.at[0, cid, ostripe])
    pltpu.sync_copy(acc2.at[ostripe], out_h.at[1, cid, ostripe])


def _cnn_body(x_ref, wfc_ref, bfc_ref, wl_ref, bl_ref, o_ref):
    xb = x_ref[...]
    y = xb[:, 2:3]
    xf = xb[:, 3:]
    h1 = jnp.maximum(
        jnp.dot(xf, wfc_ref[...], preferred_element_type=jnp.float32)
        + bfc_ref[...], 0.0)
    h2 = jnp.maximum(
        jnp.dot(h1, wl_ref[...], preferred_element_type=jnp.float32)
        + bl_ref[...], 0.0)
    o_ref[...] = jnp.concatenate([h2, y], axis=1)


def _gru_body(xg_ref, wih_ref, whh_ref, bih_ref, bhh_ref, o_ref):
    xg = xg_ref[...]
    bn = xg.shape[0]
    h = jnp.zeros((bn, 64), jnp.float32)
    for t in range(4):
        xt = xg[:, t * 41:(t + 1) * 41]
        gi = jnp.dot(xt, wih_ref[...],
                     preferred_element_type=jnp.float32) + bih_ref[...]
        gh = jnp.dot(h, whh_ref[...],
                     preferred_element_type=jnp.float32) + bhh_ref[...]
        r = jax.nn.sigmoid(gi[:, :64] + gh[:, :64])
        z = jax.nn.sigmoid(gi[:, 64:128] + gh[:, 64:128])
        nn_ = jnp.tanh(gi[:, 128:] + r * gh[:, 128:])
        h = (1.0 - z) * nn_ + z * h
    o_ref[...] = jnp.concatenate([xg[:, 164:204], h], axis=1)


def _h_body(x0_ref, degs_ref, w_ref, hs1_ref, hs2_ref, dinv_ref):
    dinv = lax.rsqrt(degs_ref[...] + 1.0)
    hb = jnp.dot(x0_ref[...], w_ref[...],
                 preferred_element_type=jnp.float32)
    hs1_ref[...] = hb[:, :80] * dinv[:, 0:1]
    hs2_ref[...] = hb[:, 40:] * dinv[:, 1:2]
    dinv_ref[...] = dinv


def _final_body(acc_ref, hs1_ref, hs2_ref, dinv_ref,
                bs1_ref, bc_ref, bs2_ref, aw1_ref, ab1_ref, aw2_ref,
                wm_ref, bm_ref,
                out_ref, beta_ref, e1_ref, c1_ref, c2_ref, e2_ref, emb_ref):
    av = acc_ref[...]
    dinv = dinv_ref[...]
    o1 = (av[0] + av[1] + hs1_ref[...]) * dinv[:, 0:1]
    o2 = (av[2] + av[3] + hs2_ref[...]) * dinv[:, 1:2]
    emb1 = jnp.maximum(o1[:, :40] + bs1_ref[...], 0.0)
    com1 = jnp.maximum(o1[:, 40:] + bc_ref[...], 0.0)
    com2 = jnp.maximum(o2[:, :40] + bc_ref[...], 0.0)
    emb2 = jnp.maximum(o2[:, 40:] + bs2_ref[...], 0.0)
    xcom = (com1 + com2) * 0.5
    aw2 = aw2_ref[...]
    ws = []
    for zb in (emb1, emb2, xcom):
        t1 = jnp.tanh(jnp.dot(zb, aw1_ref[...],
                              preferred_element_type=jnp.float32)
                      + ab1_ref[...])
        ws.append(jnp.sum(t1 * aw2, axis=1, keepdims=True))
    w = jnp.concatenate(ws, axis=1)
    wmax = jnp.max(w, axis=1, keepdims=True)
    ew = jnp.exp(w - wmax)
    beta = ew / jnp.sum(ew, axis=1, keepdims=True)
    emb = (beta[:, 0:1] * emb1 + beta[:, 1:2] * emb2 + beta[:, 2:3] * xcom)
    out_ref[...] = (jnp.sum(emb * wm_ref[...], axis=1, keepdims=True)
                    + bm_ref[...])
    beta_ref[...] = beta
    e1_ref[...] = emb1
    c1_ref[...] = com1
    c2_ref[...] = com2
    e2_ref[...] = emb2
    emb_ref[...] = emb


def _row_spec(bn, cols):
    return pl.BlockSpec((bn, cols), lambda i: (i, 0))


def _whole(shape):
    return pl.BlockSpec(shape, lambda i: tuple(0 for _ in shape))


@jax.jit
def kernel(x, edge_index, feat_edge_index, W_fc, b_fc, W_lin1, b_lin1,
           W_ih, W_hh, b_ih, b_hh, W_s1, b_s1, W_s2, b_s2, W_c, b_c,
           att_W1, att_b1, att_W2, W_mlp, b_mlp):
    f32 = jnp.float32

    x2d = x.reshape(N * 5, 395)
    R = 2000
    xcat = pl.pallas_call(
        _cnn_body,
        grid=(N * 5 // R,),
        in_specs=[_row_spec(R, 395), _whole((392, 80)), _whole((1, 80)),
                  _whole((80, 40)), _whole((1, 40))],
        out_specs=_row_spec(R, 41),
        out_shape=jax.ShapeDtypeStruct((N * 5, 41), f32),
    )(x2d, W_fc, b_fc.reshape(1, 80), W_lin1, b_lin1.reshape(1, 40))

    xg = xcat.reshape(N, 205)
    Bn = 2000
    x0 = pl.pallas_call(
        _gru_body,
        grid=(N // Bn,),
        in_specs=[_row_spec(Bn, 205), _whole((41, 192)), _whole((64, 192)),
                  _whole((1, 192)), _whole((1, 192))],
        out_specs=_row_spec(Bn, 104),
        out_shape=jax.ShapeDtypeStruct((N, 104), f32),
    )(xg, W_ih.T, W_hh.T, b_ih.reshape(1, 192), b_hh.reshape(1, 192))

    ei = edge_index.astype(jnp.int32)
    fei = feat_edge_index.astype(jnp.int32)
    pad_src = jnp.zeros((EPAD - E,), jnp.int32)
    pad_dst = jnp.full((EPAD - E,), NTRASH, jnp.int32)

    src1 = jnp.concatenate([ei[0], pad_src])
    dst1 = jnp.concatenate([ei[1], pad_dst])
    src2 = jnp.concatenate([fei[0], pad_src])
    dst2 = jnp.concatenate([fei[1], pad_dst])

    zdeg = jnp.zeros((CH, WD), f32)
    ones = jnp.ones((CH, WD), f32)
    degp = _deg_kernel_fn()(dst1, dst2, zdeg, ones)
    degs = degp.sum(axis=1)[:, :N, 0].T

    Wcat = jnp.concatenate([W_s1, W_c, W_s2], axis=1)
    hs1, hs2, dinvs = pl.pallas_call(
        _h_body,
        grid=(N // Bn,),
        in_specs=[_row_spec(Bn, 104), _row_spec(Bn, 2), _whole((104, 120))],
        out_specs=[_row_spec(Bn, 80), _row_spec(Bn, 80), _row_spec(Bn, 2)],
        out_shape=[jax.ShapeDtypeStruct((N, 80), f32),
                   jax.ShapeDtypeStruct((N, 80), f32),
                   jax.ShapeDtypeStruct((N, 2), f32)],
    )(x0, degs, Wcat)

    zacc = jnp.zeros((CH, 80), f32)
    accp = _scat_kernel_fn()(hs1, hs2, src1, dst1, src2, dst2, zacc)
    acc4 = accp.reshape(4, NACC, 80)[:, :N, :]

    accspec = pl.BlockSpec((4, Bn, 80), lambda i: (0, i, 0))
    outs = pl.pallas_call(
        _final_body,
        grid=(N // Bn,),
        in_specs=[accspec, _row_spec(Bn, 80), _row_spec(Bn, 80),
                  _row_spec(Bn, 2), _whole((1, 40)), _whole((1, 40)),
                  _whole((1, 40)), _whole((40, 16)), _whole((1, 16)),
                  _whole((1, 16)), _whole((1, 40)), _whole((1, 1))],
        out_specs=[_row_spec(Bn, 1), _row_spec(Bn, 3), _row_spec(Bn, 40),
                   _row_spec(Bn, 40), _row_spec(Bn, 40), _row_spec(Bn, 40),
                   _row_spec(Bn, 40)],
        out_shape=[jax.ShapeDtypeStruct((N, 1), f32),
                   jax.ShapeDtypeStruct((N, 3), f32),
                   jax.ShapeDtypeStruct((N, 40), f32),
                   jax.ShapeDtypeStruct((N, 40), f32),
                   jax.ShapeDtypeStruct((N, 40), f32),
                   jax.ShapeDtypeStruct((N, 40), f32),
                   jax.ShapeDtypeStruct((N, 40), f32)],
    )(acc4, hs1, hs2, dinvs,
      b_s1.reshape(1, 40), b_c.reshape(1, 40), b_s2.reshape(1, 40),
      att_W1, att_b1.reshape(1, 16), att_W2.reshape(1, 16),
      W_mlp.reshape(1, 40), b_mlp.reshape(1, 1))
    output, beta, emb1, com1, com2, emb2, emb = outs
    return (output, beta.reshape(N, 3, 1), emb1, com1, com2, emb2, emb)
